# baseline (speedup 1.0000x reference)
import functools

import jax
import jax.numpy as jnp
from jax import lax
from jax.experimental import pallas as pl
from jax.experimental.pallas import tpu as pltpu
from jax.experimental.pallas import tpu_sc as plsc

NUM_FIELDS = 26
VOCAB = 100000
EMB = 16
BATCH = 4096

_NC = 2
_NS = 16
_FPC = NUM_FIELDS // _NC
_LANES = 16

_H0 = 50048
_H1 = VOCAB - _H0
_UNROLL = 8


def _make_kernel():
    mesh = plsc.VectorSubcoreMesh(core_axis_name="c", subcore_axis_name="s")

    @functools.partial(
        pl.kernel,
        mesh=mesh,
        compiler_params=pltpu.CompilerParams(needs_layout_passes=False),
        out_type=jax.ShapeDtypeStruct((NUM_FIELDS, EMB, BATCH), jnp.float32),
        scratch_types=[
            pltpu.VMEM((_H0,), jnp.float32),
            pltpu.VMEM((_H1,), jnp.float32),
            pltpu.VMEM((BATCH,), jnp.int32),
            pltpu.VMEM((BATCH,), jnp.int32),
            pltpu.VMEM((BATCH,), jnp.float32),
            pltpu.VMEM((BATCH,), jnp.float32),
            pltpu.SemaphoreType.DMA,
            pltpu.SemaphoreType.DMA,
            pltpu.SemaphoreType.DMA,
            pltpu.SemaphoreType.DMA,
            pltpu.SemaphoreType.DMA,
            pltpu.SemaphoreType.DMA,
        ],
    )
    def k(tp_hbm, xT_hbm, out_hbm, buf0, buf1, xc0, xc1, d0, d1,
          sem0, sem1, sx0, sx1, so0, so1):
        c = lax.axis_index("c")
        s = lax.axis_index("s")
        sems = (sem0, sem1)
        bufs = (buf0, buf1)
        xcols = (xc0, xc1)
        sxs = (sx0, sx1)
        dsts = (d0, d1)
        sos = (so0, so1)
        iota = lax.iota(jnp.int32, _LANES)

        def fetch_x(fi):
            return pltpu.async_copy(
                xT_hbm.at[c * _FPC + fi], xcols[fi % 2], sxs[fi % 2]
            )

        def stage(u):
            f = c * _FPC + (u // 2)
            h = u % 2
            base = h * _H0
            n = _H1 if h else _H0
            del n
            return pltpu.async_copy(
                tp_hbm.at[f, s, pl.ds(base, _H1 if h else _H0)],
                bufs[h],
                sems[h],
            )

        pending = stage(0)
        px = fetch_x(0)
        pouts = [None, None]
        for u in range(2 * _FPC):
            f_idx = u // 2
            h = u % 2
            f = c * _FPC + f_idx
            if h == 0:
                px.wait()
                if f_idx + 1 < _FPC:
                    px = fetch_x(f_idx + 1)
                if pouts[f_idx % 2] is not None:
                    pouts[f_idx % 2].wait()
            pending.wait()
            if u + 1 < 2 * _FPC:
                pending = stage(u + 1)
            buf = bufs[h]
            xcol = xcols[f_idx % 2]
            dst = dsts[f_idx % 2]
            base = h * _H0

            def body(i, carry):
                for v in range(_UNROLL):
                    off = (i * _UNROLL + v) * _LANES
                    x = xcol[pl.ds(off, _LANES)]
                    if h == 0:
                        inb = x < _H0
                        xl = x
                    else:
                        inb = x >= _H0
                        xl = x - _H0
                    vals = plsc.load_gather(buf, [xl], mask=inb)
                    plsc.store_scatter(dst, [iota + off], vals, mask=inb)
                return carry

            lax.fori_loop(0, BATCH // (_LANES * _UNROLL), body, 0)

            if h == 1:
                pouts[f_idx % 2] = pltpu.async_copy(
                    dst, out_hbm.at[f, s], sos[f_idx % 2]
                )

        for p in pouts:
            if p is not None:
                p.wait()

    return k


_kernel_call = _make_kernel()


def kernel(X, tables):
    tp = jnp.transpose(tables, (0, 2, 1))
    xT = jnp.transpose(X, (1, 0))
    out = _kernel_call(tp, xT)
    return jnp.transpose(out, (2, 0, 1))


# speedup vs baseline: 1.0131x; 1.0131x over previous
import functools

import jax
import jax.numpy as jnp
from jax import lax
from jax.experimental import pallas as pl
from jax.experimental.pallas import tpu as pltpu
from jax.experimental.pallas import tpu_sc as plsc

NUM_FIELDS = 26
VOCAB = 100000
EMB = 16
BATCH = 4096

_NC = 2
_NS = 16
_FPC = NUM_FIELDS // _NC
_LANES = 16

_H0 = 50048
_H1 = VOCAB - _H0
_UNROLL = 4


def _make_kernel():
    mesh = plsc.VectorSubcoreMesh(core_axis_name="c", subcore_axis_name="s")

    @functools.partial(
        pl.kernel,
        mesh=mesh,
        compiler_params=pltpu.CompilerParams(needs_layout_passes=False),
        out_type=jax.ShapeDtypeStruct((NUM_FIELDS, EMB, BATCH), jnp.float32),
        scratch_types=[
            pltpu.VMEM((_H0,), jnp.float32),
            pltpu.VMEM((_H1,), jnp.float32),
            pltpu.VMEM((BATCH,), jnp.int32),
            pltpu.VMEM((BATCH,), jnp.int32),
            pltpu.VMEM((BATCH,), jnp.float32),
            pltpu.VMEM((BATCH,), jnp.float32),
            pltpu.SemaphoreType.DMA,
            pltpu.SemaphoreType.DMA,
            pltpu.SemaphoreType.DMA,
            pltpu.SemaphoreType.DMA,
            pltpu.SemaphoreType.DMA,
            pltpu.SemaphoreType.DMA,
        ],
    )
    def k(tp_hbm, xT_hbm, out_hbm, buf0, buf1, xc0, xc1, d0, d1,
          sem0, sem1, sx0, sx1, so0, so1):
        c = lax.axis_index("c")
        s = lax.axis_index("s")
        sems = (sem0, sem1)
        bufs = (buf0, buf1)
        xcols = (xc0, xc1)
        sxs = (sx0, sx1)
        dsts = (d0, d1)
        sos = (so0, so1)
        iota = lax.iota(jnp.int32, _LANES)

        def fetch_x(fi):
            return pltpu.async_copy(
                xT_hbm.at[c * _FPC + fi], xcols[fi % 2], sxs[fi % 2]
            )

        def stage(u):
            f = c * _FPC + (u // 2)
            h = u % 2
            base = h * _H0
            n = _H1 if h else _H0
            del n
            return pltpu.async_copy(
                tp_hbm.at[f, s, pl.ds(base, _H1 if h else _H0)],
                bufs[h],
                sems[h],
            )

        pending = stage(0)
        px = fetch_x(0)
        pouts = [None, None]
        for u in range(2 * _FPC):
            f_idx = u // 2
            h = u % 2
            f = c * _FPC + f_idx
            if h == 0:
                px.wait()
                if f_idx + 1 < _FPC:
                    px = fetch_x(f_idx + 1)
                if pouts[f_idx % 2] is not None:
                    pouts[f_idx % 2].wait()
            nxt = stage(u + 1) if u + 1 < 2 * _FPC else None
            pending.wait()
            buf = bufs[h]
            xcol = xcols[f_idx % 2]
            dst = dsts[f_idx % 2]
            base = h * _H0

            def body(i, carry):
                for v in range(_UNROLL):
                    off = (i * _UNROLL + v) * _LANES
                    x = xcol[pl.ds(off, _LANES)]
                    if h == 0:
                        inb = x < _H0
                        xl = x
                    else:
                        inb = x >= _H0
                        xl = x - _H0
                    vals = plsc.load_gather(buf, [xl], mask=inb)
                    plsc.store_scatter(dst, [iota + off], vals, mask=inb)
                return carry

            lax.fori_loop(0, BATCH // (_LANES * _UNROLL), body, 0)

            if h == 1:
                pouts[f_idx % 2] = pltpu.async_copy(
                    dst, out_hbm.at[f, s], sos[f_idx % 2]
                )
            pending = nxt

        for p in pouts:
            if p is not None:
                p.wait()

    return k


_kernel_call = _make_kernel()


def kernel(X, tables):
    tp = jnp.transpose(tables, (0, 2, 1))
    xT = jnp.transpose(X, (1, 0))
    out = _kernel_call(tp, xT)
    return jnp.transpose(out, (2, 0, 1))
